# baseline (speedup 1.0000x reference)
import math

import jax
import jax.numpy as jnp
from jax.experimental import pallas as pl
from jax.experimental.pallas import tpu as pltpu

_B, _N, _D = 32, 577, 384
_H, _HD = 6, 64
_R = 16
_HID = _D * 4
_NP = (_N - 1) // 2
_NU = _NP - _R
_NM = _N - _R
_PT = 576
_HP = jax.lax.Precision.HIGHEST


def _ln(x, g, b, eps=1e-6):
    m = jnp.mean(x, axis=-1, keepdims=True)
    v = jnp.mean((x - m) * (x - m), axis=-1, keepdims=True)
    return (x - m) * jax.lax.rsqrt(v + eps) * g + b


def _gelu(x):
    return 0.5 * x * (1.0 + jax.lax.erf(x * (1.0 / math.sqrt(2.0))))


def _tome_kernel(xr_ref, clsp_ref, meta_ref, g1_ref, b1_ref,
                 qkvw_ref, qkvb_ref, projw_ref, projb_ref, g2_ref, b2_ref,
                 fc1w_ref, fc1b_ref, fc2w_ref, fc2b_ref, out_ref):
    xr = xr_ref[0]
    dstx = xr[:, :_D]
    srcx = xr[:, _D:]
    meta = meta_ref[0]
    nidx = meta[0:1]
    mrow = meta[1:2]
    pos = meta[2:3]

    ridx = jax.lax.broadcasted_iota(jnp.int32, (_NP, _NP), 0)
    nidx_i = nidx.astype(jnp.int32)
    pos_i = pos.astype(jnp.int32)
    merged = mrow == 1.0

    S = jnp.where((nidx_i == ridx) & merged, 1.0, 0.0)
    cnt = 1.0 + jnp.sum(S, axis=1, keepdims=True)
    rc = 1.0 / cnt
    rc = rc * (2.0 - cnt * rc)
    dst_tok = (dstx + jnp.dot(S, srcx, preferred_element_type=jnp.float32,
                              precision=_HP)) * rc

    P = jnp.where((pos_i == ridx) & (~merged), 1.0, 0.0)
    unm = jnp.dot(P, srcx, preferred_element_type=jnp.float32,
                  precision=_HP)

    t2 = jnp.concatenate([dst_tok, unm[:_NU], clsp_ref[0]], axis=0)

    g1 = g1_ref[...]
    b1 = b1_ref[...]
    xn2 = _ln(t2, g1, b1)
    qkv = jnp.dot(xn2, qkvw_ref[...], preferred_element_type=jnp.float32,
                  precision=_HP) + qkvb_ref[...]
    kmask = jax.lax.broadcasted_iota(jnp.int32, (_PT, _PT), 1) < _NM
    scale = _HD ** -0.5
    outs = []
    for h in range(_H):
        q = jax.lax.slice(qkv, (0, h * _HD), (_PT, (h + 1) * _HD))
        k = jax.lax.slice(qkv, (0, _D + h * _HD), (_PT, _D + (h + 1) * _HD))
        v = jax.lax.slice(qkv, (0, 2 * _D + h * _HD),
                          (_PT, 2 * _D + (h + 1) * _HD))
        s = jax.lax.dot_general(q, k, (((1,), (1,)), ((), ())),
                                preferred_element_type=jnp.float32,
                                precision=_HP) * scale
        s = jnp.where(kmask, s, -1e30)
        s = s - jnp.max(s, axis=1, keepdims=True)
        e = jnp.exp(s)
        p = e / jnp.sum(e, axis=1, keepdims=True)
        outs.append(jnp.dot(p, v, preferred_element_type=jnp.float32,
                            precision=_HP))
    att = jnp.concatenate(outs, axis=1)

    x2 = t2 + jnp.dot(att, projw_ref[...], preferred_element_type=jnp.float32,
                      precision=_HP) + projb_ref[...]
    hh = jnp.dot(_ln(x2, g2_ref[...], b2_ref[...]), fc1w_ref[...],
                 preferred_element_type=jnp.float32, precision=_HP) \
        + fc1b_ref[...]
    hh = _gelu(hh)
    out_ref[0] = x2 + jnp.dot(hh, fc2w_ref[...],
                              preferred_element_type=jnp.float32,
                              precision=_HP) + fc2b_ref[...]


@jax.jit
def kernel(x, ln1_g, ln1_b, qkv_w, qkv_b, proj_w, proj_b, ln2_g, ln2_b,
           fc1_w, fc1_b, fc2_w, fc2_b):
    b = x.shape[0]
    xn = _ln(x, ln1_g, ln1_b)
    qkv0 = (xn @ qkv_w + qkv_b).reshape(b, _N, 3, _H, _HD)
    pk = qkv0[:, :, 1].mean(axis=2)[:, 1:]
    nrm = jnp.linalg.norm(pk, axis=-1, keepdims=True)
    metric = pk / jnp.maximum(nrm, 1e-12)
    scores = jnp.einsum('bsc,bdc->bsd', metric[:, 1::2], metric[:, 0::2])
    node_max = scores.max(axis=-1)
    node_idx = scores.argmax(axis=-1)
    _, edge_idx = jax.lax.top_k(node_max, _R)
    mask = jnp.zeros((b, _NP), jnp.float32).at[
        jnp.arange(b)[:, None], edge_idx].set(1.0)
    inv = 1.0 - mask
    pos = jnp.cumsum(inv, axis=1) - inv
    meta = jnp.stack([node_idx.astype(jnp.float32), mask, pos], axis=1)
    meta = jnp.pad(meta, ((0, 0), (0, 5), (0, 0)))

    xr = x[:, 1:, :].reshape(b, _NP, 2 * _D)
    clsp = jnp.concatenate(
        [x[:, :1, :], jnp.zeros((b, 15, _D), x.dtype)], axis=1)

    def cspec(shape):
        return pl.BlockSpec(shape, lambda i: tuple(0 for _ in shape))

    out = pl.pallas_call(
        _tome_kernel,
        grid=(b,),
        in_specs=[
            pl.BlockSpec((1, _NP, 2 * _D), lambda i: (i, 0, 0)),
            pl.BlockSpec((1, 16, _D), lambda i: (i, 0, 0)),
            pl.BlockSpec((1, 8, _NP), lambda i: (i, 0, 0)),
            cspec((_D,)),
            cspec((_D,)),
            cspec((_D, 3 * _D)),
            cspec((3 * _D,)),
            cspec((_D, _D)),
            cspec((_D,)),
            cspec((_D,)),
            cspec((_D,)),
            cspec((_D, _HID)),
            cspec((_HID,)),
            cspec((_HID, _D)),
            cspec((_D,)),
        ],
        out_specs=pl.BlockSpec((1, _PT, _D), lambda i: (i, 0, 0)),
        out_shape=jax.ShapeDtypeStruct((b, _PT, _D), jnp.float32),
        compiler_params=pltpu.CompilerParams(
            dimension_semantics=("arbitrary",),
        ),
    )(xr, clsp, meta, ln1_g, ln1_b, qkv_w, qkv_b, proj_w, proj_b,
      ln2_g, ln2_b, fc1_w, fc1_b, fc2_w, fc2_b)

    return jnp.concatenate([out[:, _PT - 16:_PT - 15], out[:, :_NM - 1]],
                           axis=1)


# speedup vs baseline: 2.1739x; 2.1739x over previous
import math

import jax
import jax.numpy as jnp
from jax.experimental import pallas as pl
from jax.experimental.pallas import tpu as pltpu

_B, _N, _D = 32, 577, 384
_H, _HD = 6, 64
_R = 16
_HID = _D * 4
_NP = (_N - 1) // 2
_NU = _NP - _R
_NM = _N - _R
_PT = 576
_HP = jax.lax.Precision.HIGHEST


def _ln(x, g, b, eps=1e-6):
    m = jnp.mean(x, axis=-1, keepdims=True)
    v = jnp.mean((x - m) * (x - m), axis=-1, keepdims=True)
    return (x - m) * jax.lax.rsqrt(v + eps) * g + b


def _gelu(x):
    return 0.5 * x * (1.0 + jax.lax.erf(x * (1.0 / math.sqrt(2.0))))


def _tome_kernel(xr_ref, clsp_ref, meta_ref, g1_ref, b1_ref,
                 qkvw_ref, qkvb_ref, projw_ref, projb_ref, g2_ref, b2_ref,
                 fc1w_ref, fc1b_ref, fc2w_ref, fc2b_ref, out_ref):
    xr = xr_ref[0]
    dstx = xr[:, :_D]
    srcx = xr[:, _D:]
    meta = meta_ref[0]
    nidx = meta[0:1]
    mrow = meta[1:2]
    pos = meta[2:3]

    ridx = jax.lax.broadcasted_iota(jnp.int32, (_NP, _NP), 0)
    nidx_i = nidx.astype(jnp.int32)
    pos_i = pos.astype(jnp.int32)
    merged = mrow == 1.0

    S = jnp.where((nidx_i == ridx) & merged, 1.0, 0.0)
    cnt = 1.0 + jnp.sum(S, axis=1, keepdims=True)
    rc = 1.0 / cnt
    rc = rc * (2.0 - cnt * rc)
    dst_tok = (dstx + jnp.dot(S, srcx, preferred_element_type=jnp.float32,
                              precision=_HP)) * rc

    P = jnp.where((pos_i == ridx) & (~merged), 1.0, 0.0)
    unm = jnp.dot(P, srcx, preferred_element_type=jnp.float32,
                  precision=_HP)

    t2 = jnp.concatenate([dst_tok, unm[:_NU], clsp_ref[0]], axis=0)

    g1 = g1_ref[...]
    b1 = b1_ref[...]
    xn2 = _ln(t2, g1, b1)
    qkv = jnp.dot(xn2, qkvw_ref[...],
                  preferred_element_type=jnp.float32) + qkvb_ref[...]
    kmask = jax.lax.broadcasted_iota(jnp.int32, (_PT, _PT), 1) < _NM
    scale = _HD ** -0.5
    outs = []
    for h in range(_H):
        q = jax.lax.slice(qkv, (0, h * _HD), (_PT, (h + 1) * _HD))
        k = jax.lax.slice(qkv, (0, _D + h * _HD), (_PT, _D + (h + 1) * _HD))
        v = jax.lax.slice(qkv, (0, 2 * _D + h * _HD),
                          (_PT, 2 * _D + (h + 1) * _HD))
        s = jax.lax.dot_general(q, k, (((1,), (1,)), ((), ())),
                                preferred_element_type=jnp.float32) * scale
        s = jnp.where(kmask, s, -1e30)
        s = s - jnp.max(s, axis=1, keepdims=True)
        e = jnp.exp(s)
        p = e / jnp.sum(e, axis=1, keepdims=True)
        outs.append(jnp.dot(p, v, preferred_element_type=jnp.float32))
    att = jnp.concatenate(outs, axis=1)

    x2 = t2 + jnp.dot(att, projw_ref[...],
                      preferred_element_type=jnp.float32) + projb_ref[...]
    hh = jnp.dot(_ln(x2, g2_ref[...], b2_ref[...]), fc1w_ref[...],
                 preferred_element_type=jnp.float32) + fc1b_ref[...]
    hh = _gelu(hh)
    out_ref[0] = x2 + jnp.dot(hh, fc2w_ref[...],
                              preferred_element_type=jnp.float32) + fc2b_ref[...]


@jax.jit
def kernel(x, ln1_g, ln1_b, qkv_w, qkv_b, proj_w, proj_b, ln2_g, ln2_b,
           fc1_w, fc1_b, fc2_w, fc2_b):
    b = x.shape[0]
    xn = _ln(x, ln1_g, ln1_b)
    qkv0 = (xn @ qkv_w + qkv_b).reshape(b, _N, 3, _H, _HD)
    pk = qkv0[:, :, 1].mean(axis=2)[:, 1:]
    nrm = jnp.linalg.norm(pk, axis=-1, keepdims=True)
    metric = pk / jnp.maximum(nrm, 1e-12)
    scores = jnp.einsum('bsc,bdc->bsd', metric[:, 1::2], metric[:, 0::2])
    node_max = scores.max(axis=-1)
    node_idx = scores.argmax(axis=-1)
    _, edge_idx = jax.lax.top_k(node_max, _R)
    mask = jnp.zeros((b, _NP), jnp.float32).at[
        jnp.arange(b)[:, None], edge_idx].set(1.0)
    inv = 1.0 - mask
    pos = jnp.cumsum(inv, axis=1) - inv
    meta = jnp.stack([node_idx.astype(jnp.float32), mask, pos], axis=1)
    meta = jnp.pad(meta, ((0, 0), (0, 5), (0, 0)))

    xr = x[:, 1:, :].reshape(b, _NP, 2 * _D)
    clsp = jnp.concatenate(
        [x[:, :1, :], jnp.zeros((b, 15, _D), x.dtype)], axis=1)

    def cspec(shape):
        return pl.BlockSpec(shape, lambda i: tuple(0 for _ in shape))

    out = pl.pallas_call(
        _tome_kernel,
        grid=(b,),
        in_specs=[
            pl.BlockSpec((1, _NP, 2 * _D), lambda i: (i, 0, 0)),
            pl.BlockSpec((1, 16, _D), lambda i: (i, 0, 0)),
            pl.BlockSpec((1, 8, _NP), lambda i: (i, 0, 0)),
            cspec((_D,)),
            cspec((_D,)),
            cspec((_D, 3 * _D)),
            cspec((3 * _D,)),
            cspec((_D, _D)),
            cspec((_D,)),
            cspec((_D,)),
            cspec((_D,)),
            cspec((_D, _HID)),
            cspec((_HID,)),
            cspec((_HID, _D)),
            cspec((_D,)),
        ],
        out_specs=pl.BlockSpec((1, _PT, _D), lambda i: (i, 0, 0)),
        out_shape=jax.ShapeDtypeStruct((b, _PT, _D), jnp.float32),
        compiler_params=pltpu.CompilerParams(
            dimension_semantics=("arbitrary",),
        ),
    )(xr, clsp, meta, ln1_g, ln1_b, qkv_w, qkv_b, proj_w, proj_b,
      ln2_g, ln2_b, fc1_w, fc1_b, fc2_w, fc2_b)

    return jnp.concatenate([out[:, _PT - 16:_PT - 15], out[:, :_NM - 1]],
                           axis=1)
